# baseline (speedup 1.0000x reference)
import functools

import jax
import jax.numpy as jnp
import numpy as np
from jax import lax
from jax.experimental import pallas as pl
from jax.experimental.pallas import tpu as pltpu
from jax.experimental.pallas import tpu_sc as plsc

_T, _B, _D = 576, 64, 768
_REMAIN = 144
_NROWS = _REMAIN * _B
_NW = 32
_ROWS_PER_W = _NROWS // _NW
_CHUNK = 72
_NCHUNK = _ROWS_PER_W // _CHUNK
_NBUF = 2


def _constant_indexes():
    def build():
        keys = jax.random.split(jax.random.key(42), _B)
        perms = [jax.random.permutation(k, _T) for k in keys]
        fwd = jnp.stack(perms, axis=-1).astype(jnp.int32)
        bwd = jnp.argsort(fwd, axis=0).astype(jnp.int32)
        return fwd, bwd

    fwd, bwd = jax.jit(build)()
    return np.asarray(fwd), np.asarray(bwd)


_FWD, _BWD = _constant_indexes()
_FLAT_IDX = (
    (_FWD[:_REMAIN] * _B + np.arange(_B, dtype=np.int32)[None, :])
    .astype(np.int32)
    .reshape(_NW, _NCHUNK, _CHUNK)
)


@functools.cache
def _build_gather():
    @functools.partial(
        pl.kernel,
        out_type=jax.ShapeDtypeStruct((_NROWS, _D), jnp.float32),
        mesh=plsc.VectorSubcoreMesh(core_axis_name="c", subcore_axis_name="s"),
        scratch_types=(
            [pltpu.VMEM((_NCHUNK, _CHUNK), jnp.int32)]
            + [pltpu.VMEM((_CHUNK, _D), jnp.float32)] * _NBUF
            + [pltpu.SemaphoreType.DMA] * (2 * _NBUF)
        ),
    )
    def _gather_rows(src_hbm, idx_hbm, out_hbm, idx_v, *scr):
        wid = lax.axis_index("s") * 2 + lax.axis_index("c")
        base = wid * _ROWS_PER_W
        pltpu.sync_copy(idx_hbm.at[wid], idx_v)
        bufs = scr[:_NBUF]
        gsems = scr[_NBUF : 2 * _NBUF]
        wsems = scr[2 * _NBUF :]
        g = [None] * _NCHUNK
        w = [None] * _NCHUNK
        for j in range(_NBUF - 1):
            g[j] = pltpu.async_copy(src_hbm.at[idx_v.at[j]], bufs[j], gsems[j])
        for j in range(_NCHUNK):
            g[j].wait()
            w[j] = pltpu.async_copy(
                bufs[j % _NBUF],
                out_hbm.at[pl.ds(base + j * _CHUNK, _CHUNK)],
                wsems[j % _NBUF],
            )
            nxt = j + _NBUF - 1
            if nxt < _NCHUNK:
                if nxt - _NBUF >= 0:
                    w[nxt - _NBUF].wait()
                g[nxt] = pltpu.async_copy(
                    src_hbm.at[idx_v.at[nxt]], bufs[nxt % _NBUF], gsems[nxt % _NBUF]
                )
        for j in range(_NCHUNK - _NBUF, _NCHUNK):
            w[j].wait()

    return _gather_rows


def kernel(patches):
    src = patches.reshape(_T * _B, _D)
    out = _build_gather()(src, jnp.asarray(_FLAT_IDX))
    return (
        out.reshape(_REMAIN, _B, _D),
        jnp.asarray(_FWD),
        jnp.asarray(_BWD),
    )


# speedup vs baseline: 1.0281x; 1.0281x over previous
import functools

import jax
import jax.numpy as jnp
import numpy as np
from jax import lax
from jax.experimental import pallas as pl
from jax.experimental.pallas import tpu as pltpu
from jax.experimental.pallas import tpu_sc as plsc

_T, _B, _D = 576, 64, 768
_REMAIN = 144
_NROWS = _REMAIN * _B
_NW = 32
_ROWS_PER_W = _NROWS // _NW
_CHUNK = 48
_NCHUNK = _ROWS_PER_W // _CHUNK
_NBUF = 3


def _constant_indexes():
    def build():
        keys = jax.random.split(jax.random.key(42), _B)
        perms = [jax.random.permutation(k, _T) for k in keys]
        fwd = jnp.stack(perms, axis=-1).astype(jnp.int32)
        bwd = jnp.argsort(fwd, axis=0).astype(jnp.int32)
        return fwd, bwd

    fwd, bwd = jax.jit(build)()
    return np.asarray(fwd), np.asarray(bwd)


_FWD, _BWD = _constant_indexes()
_FLAT_IDX = (
    (_FWD[:_REMAIN] * _B + np.arange(_B, dtype=np.int32)[None, :])
    .astype(np.int32)
    .reshape(_NW, _NCHUNK, _CHUNK)
)


@functools.cache
def _build_gather():
    @functools.partial(
        pl.kernel,
        out_type=jax.ShapeDtypeStruct((_NROWS, _D), jnp.float32),
        mesh=plsc.VectorSubcoreMesh(core_axis_name="c", subcore_axis_name="s"),
        scratch_types=(
            [pltpu.VMEM((_NCHUNK, _CHUNK), jnp.int32)]
            + [pltpu.VMEM((_CHUNK, _D), jnp.float32)] * _NBUF
            + [pltpu.SemaphoreType.DMA] * (2 * _NBUF)
        ),
    )
    def _gather_rows(src_hbm, idx_hbm, out_hbm, idx_v, *scr):
        wid = lax.axis_index("s") * 2 + lax.axis_index("c")
        base = wid * _ROWS_PER_W
        pltpu.sync_copy(idx_hbm.at[wid], idx_v)
        bufs = scr[:_NBUF]
        gsems = scr[_NBUF : 2 * _NBUF]
        wsems = scr[2 * _NBUF :]
        g = [None] * _NCHUNK
        w = [None] * _NCHUNK
        for j in range(_NBUF - 1):
            g[j] = pltpu.async_copy(src_hbm.at[idx_v.at[j]], bufs[j], gsems[j])
        for j in range(_NCHUNK):
            g[j].wait()
            w[j] = pltpu.async_copy(
                bufs[j % _NBUF],
                out_hbm.at[pl.ds(base + j * _CHUNK, _CHUNK)],
                wsems[j % _NBUF],
            )
            nxt = j + _NBUF - 1
            if nxt < _NCHUNK:
                if nxt - _NBUF >= 0:
                    w[nxt - _NBUF].wait()
                g[nxt] = pltpu.async_copy(
                    src_hbm.at[idx_v.at[nxt]], bufs[nxt % _NBUF], gsems[nxt % _NBUF]
                )
        for j in range(_NCHUNK - _NBUF, _NCHUNK):
            w[j].wait()

    return _gather_rows


def kernel(patches):
    src = patches.reshape(_T * _B, _D)
    out = _build_gather()(src, jnp.asarray(_FLAT_IDX))
    return (
        out.reshape(_REMAIN, _B, _D),
        jnp.asarray(_FWD),
        jnp.asarray(_BWD),
    )


# speedup vs baseline: 1.0285x; 1.0004x over previous
import functools

import jax
import jax.numpy as jnp
import numpy as np
from jax import lax
from jax.experimental import pallas as pl
from jax.experimental.pallas import tpu as pltpu
from jax.experimental.pallas import tpu_sc as plsc

_T, _B, _D = 576, 64, 768
_REMAIN = 144
_NROWS = _REMAIN * _B
_NW = 32
_ROWS_PER_W = _NROWS // _NW
_CHUNK = 48
_NCHUNK = _ROWS_PER_W // _CHUNK
_NBUF = 3


def _constant_indexes():
    def build():
        keys = jax.random.split(jax.random.key(42), _B)
        perms = [jax.random.permutation(k, _T) for k in keys]
        fwd = jnp.stack(perms, axis=-1).astype(jnp.int32)
        bwd = jnp.argsort(fwd, axis=0).astype(jnp.int32)
        return fwd, bwd

    fwd, bwd = jax.jit(build)()
    return np.asarray(fwd), np.asarray(bwd)


_FWD, _BWD = _constant_indexes()
_FLAT_IDX = (
    (_FWD[:_REMAIN] * _B + np.arange(_B, dtype=np.int32)[None, :])
    .astype(np.int32)
    .reshape(_NW, _NCHUNK, _CHUNK)
)


@functools.cache
def _build_gather():
    @functools.partial(
        pl.kernel,
        out_type=jax.ShapeDtypeStruct((_NROWS, _D), jnp.float32),
        mesh=plsc.VectorSubcoreMesh(core_axis_name="c", subcore_axis_name="s"),
        scratch_types=(
            [pltpu.VMEM((_NCHUNK, _CHUNK), jnp.int32)]
            + [pltpu.VMEM((_CHUNK, _D), jnp.float32)] * _NBUF
            + [pltpu.SemaphoreType.DMA] * (2 * _NBUF)
        ),
    )
    def _gather_rows(src_hbm, idx_hbm, out_hbm, idx_v, *scr):
        wid = lax.axis_index("s") * 2 + lax.axis_index("c")
        base = wid * _ROWS_PER_W
        pltpu.sync_copy(idx_hbm.at[wid], idx_v)
        bufs = scr[:_NBUF]
        gsems = scr[_NBUF : 2 * _NBUF]
        wsems = scr[2 * _NBUF :]
        g = [None] * _NCHUNK
        w = [None] * _NCHUNK
        for j in range(_NBUF - 1):
            g[j] = pltpu.async_copy(src_hbm.at[idx_v.at[j]], bufs[j], gsems[j])
        for j in range(_NCHUNK):
            g[j].wait()
            w[j] = pltpu.async_copy(
                bufs[j % _NBUF],
                out_hbm.at[pl.ds(base + j * _CHUNK, _CHUNK)],
                wsems[j % _NBUF],
            )
            nxt = j + _NBUF - 1
            if nxt < _NCHUNK:
                if nxt - _NBUF >= 0:
                    w[nxt - _NBUF].wait()
                g[nxt] = pltpu.async_copy(
                    src_hbm.at[idx_v.at[nxt]], bufs[nxt % _NBUF], gsems[nxt % _NBUF]
                )
        for j in range(_NCHUNK - _NBUF, _NCHUNK):
            w[j].wait()

    return _gather_rows


def kernel(patches):
    src = patches.reshape(_T * _B, _D)
    out = _build_gather()(src, jnp.asarray(_FLAT_IDX))
    idx_dtype = jnp.int64 if jax.config.jax_enable_x64 else jnp.int32
    return (
        out.reshape(_REMAIN, _B, _D),
        jnp.asarray(_FWD, dtype=idx_dtype),
        jnp.asarray(_BWD, dtype=idx_dtype),
    )
